# baseline (speedup 1.0000x reference)
import jax
import jax.numpy as jnp
from jax import lax
from jax.experimental import pallas as pl
from jax.experimental.pallas import tpu as pltpu
from jax.experimental.pallas import tpu_sc as plsc

_REQ_R = 3840
_K = 256
_T = 4096
_C = 768
_HIGH = jax.lax.Precision.HIGHEST


def _assign_body(x_ref, base_ref, nrm_ref, assign_ref, inv_ref):
    T, C, K = _T, _C, _K
    x = x_ref[0]
    base = base_ref[0, 0]
    nrm = nrm_ref[0, 0]

    feat = x / nrm[:, None]

    tidx = jax.lax.iota(jnp.int32, T).astype(jnp.float32)
    sel = jnp.where(tidx == 0.0, jnp.inf, base)

    TB = 256
    parts = []
    for i in range(T // TB):
        st = sel[i * TB:(i + 1) * TB]
        cnt = jnp.zeros((TB,), jnp.float32)
        if i > 0:
            lo = sel[:i * TB]
            cnt = cnt + jnp.sum(
                jnp.where(lo[None, :] >= st[:, None], 1.0, 0.0), axis=1)
        if i < T // TB - 1:
            hi = sel[(i + 1) * TB:]
            cnt = cnt + jnp.sum(
                jnp.where(hi[None, :] > st[:, None], 1.0, 0.0), axis=1)
        dg = sel[i * TB:(i + 1) * TB]
        dl = jax.lax.iota(jnp.int32, TB).astype(jnp.float32)
        gt = dg[None, :] > st[:, None]
        ge = (dg[None, :] >= st[:, None]) & (dl[None, :] < dl[:, None])
        cnt = cnt + jnp.sum(jnp.where(gt | ge, 1.0, 0.0), axis=1)
        parts.append(cnt)
    rank = jnp.concatenate(parts)

    kidx = jax.lax.iota(jnp.int32, K).astype(jnp.float32)
    P = jnp.where(rank[None, :] == kidx[:, None], 1.0, 0.0)
    centers = jax.lax.dot_general(P, feat, (((1,), (0,)), ((), ())),
                                  preferred_element_type=jnp.float32)

    sims = jax.lax.dot_general(feat, centers, (((1,), (1,)), ((), ())),
                               preferred_element_type=jnp.float32)
    mx = jnp.max(sims, axis=1)
    am = jnp.min(jnp.where(sims == mx[:, None], kidx[None, :], float(K)),
                 axis=1)
    assign = jnp.where(rank < float(K), rank, am)

    A = jnp.where(assign[None, :] == kidx[:, None], 1.0, 0.0)
    sizes = jnp.sum(A, axis=1)
    inv_ref[0, 0] = 1.0 / jnp.clip(sizes, 1.0, None)
    assign_ref[0, 0] = assign.astype(jnp.int32)


def _tc_assign(x, base, nrm):
    B, T, C = x.shape
    K = _K
    return pl.pallas_call(
        _assign_body,
        grid=(B,),
        in_specs=[pl.BlockSpec((1, T, C), lambda b: (b, 0, 0)),
                  pl.BlockSpec((1, 1, T), lambda b: (b, 0, 0)),
                  pl.BlockSpec((1, 1, T), lambda b: (b, 0, 0))],
        out_specs=[pl.BlockSpec((1, 1, T), lambda b: (b, 0, 0)),
                   pl.BlockSpec((1, 1, K), lambda b: (b, 0, 0))],
        out_shape=[jax.ShapeDtypeStruct((B, 1, T), jnp.int32),
                   jax.ShapeDtypeStruct((B, 1, K), jnp.float32)],
    )(x, base.reshape(B, 1, T), nrm.reshape(B, 1, T))


_CSL = 128
_TCH = 128


def _merge_body(x_ref, aidx_ref, inv_ref, out_ref,
                rows_a, rows_b, idx_a, idx_b, acc_v, inv_v, sem_a, sem_b):
    C, K, T = _C, _K, _T
    c = lax.axis_index("c")
    s = lax.axis_index("s")
    w = c * 16 + s
    b = w // 6
    c0 = (w % 6) * _CSL

    @pl.when(w < 24)
    def _active():
        zv = jnp.zeros((16,), jnp.float32)

        def _zero_row(r, _):
            for v in range(_CSL // 16):
                acc_v[r, pl.ds(v * 16, 16)] = zv
            return 0

        lax.fori_loop(0, K, _zero_row, 0)

        def _start(i, rows, idx, sem):
            off = b * T + i * _TCH
            pltpu.async_copy(
                x_ref.at[pl.ds(off, _TCH), pl.ds(c0, _CSL)], rows, sem)
            pltpu.async_copy(aidx_ref.at[pl.ds(off, _TCH)], idx, sem)

        def _drain(i, rows, idx, sem):
            off = b * T + i * _TCH
            pltpu.make_async_copy(
                x_ref.at[pl.ds(off, _TCH), pl.ds(c0, _CSL)], rows, sem).wait()
            pltpu.make_async_copy(aidx_ref.at[pl.ds(off, _TCH)], idx, sem).wait()

        def _consume(rows, idx):
            for g in range(_TCH // 16):
                idxvec = idx[pl.ds(g * 16, 16)]
                ks = [idxvec[j] for j in range(16)]
                for j in range(16):
                    for v in range(_CSL // 16):
                        sl = pl.ds(v * 16, 16)
                        plsc.addupdate(acc_v.at[ks[j], sl],
                                       rows[g * 16 + j, sl])

        _start(0, rows_a, idx_a, sem_a)

        def _pair(p, _):
            i = 2 * p
            _start(i + 1, rows_b, idx_b, sem_b)
            _drain(i, rows_a, idx_a, sem_a)
            _consume(rows_a, idx_a)

            @pl.when(i + 2 < T // _TCH)
            def _nxt():
                _start(i + 2, rows_a, idx_a, sem_a)

            _drain(i + 1, rows_b, idx_b, sem_b)
            _consume(rows_b, idx_b)
            return 0

        lax.fori_loop(0, T // _TCH // 2, _pair, 0)

        pltpu.sync_copy(inv_ref.at[pl.ds(b * K, K), :], inv_v)

        def _scale_row(r, _):
            iv = inv_v[r, pl.ds(0, 16)]
            for v in range(_CSL // 16):
                sl = pl.ds(v * 16, 16)
                acc_v[r, sl] = acc_v[r, sl] * iv
            return 0

        lax.fori_loop(0, K, _scale_row, 0)
        pltpu.sync_copy(acc_v, out_ref.at[pl.ds(b * K, K), pl.ds(c0, _CSL)])


def _sc_merge(x_flat, aidx_flat, inv_flat):
    BT, C = x_flat.shape
    BK = inv_flat.shape[0]
    inv16 = jnp.broadcast_to(inv_flat[:, None], (BK, 16))
    mesh = plsc.VectorSubcoreMesh(core_axis_name="c", subcore_axis_name="s")
    f = pl.kernel(
        _merge_body,
        out_type=jax.ShapeDtypeStruct((BK, C), jnp.float32),
        mesh=mesh,
        scratch_types=[
            pltpu.VMEM((_TCH, _CSL), jnp.float32),
            pltpu.VMEM((_TCH, _CSL), jnp.float32),
            pltpu.VMEM((_TCH,), jnp.int32),
            pltpu.VMEM((_TCH,), jnp.int32),
            pltpu.VMEM((256, _CSL), jnp.float32),
            pltpu.VMEM((256, 16), jnp.float32),
            pltpu.SemaphoreType.DMA,
            pltpu.SemaphoreType.DMA,
        ],
    )
    return f(x_flat, aidx_flat, inv16)


def kernel(x, layer_idx, requested_r):
    B, T, C = x.shape
    K = max(1, T - _REQ_R)
    sumsq = jnp.sum(x * x, axis=-1)
    base = jnp.sqrt(sumsq + 1e-6)
    nrm = jnp.clip(jnp.linalg.norm(x, axis=-1), 1e-12, None)
    assign3, inv3 = _tc_assign(x, base, nrm)
    out_flat = _sc_merge(x.reshape(B * T, C),
                         assign3.reshape(B * T),
                         inv3.reshape(B * K))
    return out_flat.reshape(B, K, C)


# speedup vs baseline: 1.1532x; 1.1532x over previous
import jax
import jax.numpy as jnp
from jax import lax
from jax.experimental import pallas as pl
from jax.experimental.pallas import tpu as pltpu
from jax.experimental.pallas import tpu_sc as plsc

_REQ_R = 3840
_K = 256
_T = 4096
_C = 768
_HIGH = jax.lax.Precision.HIGHEST


def _assign_body(x_ref, base_ref, nrm_ref, assign_ref, inv_ref):
    T, C, K = _T, _C, _K
    x = x_ref[0]
    base = base_ref[0, 0]
    nrm = nrm_ref[0, 0]

    feat = x / nrm[:, None]

    tidx = jax.lax.iota(jnp.int32, T).astype(jnp.float32)
    sel = jnp.where(tidx == 0.0, jnp.inf, base)

    TB = 256
    parts = []
    for i in range(T // TB):
        st = sel[i * TB:(i + 1) * TB]
        cnt = jnp.zeros((TB,), jnp.float32)
        if i > 0:
            lo = sel[:i * TB]
            cnt = cnt + jnp.sum(
                jnp.where(lo[None, :] >= st[:, None], 1.0, 0.0), axis=1)
        if i < T // TB - 1:
            hi = sel[(i + 1) * TB:]
            cnt = cnt + jnp.sum(
                jnp.where(hi[None, :] > st[:, None], 1.0, 0.0), axis=1)
        dg = sel[i * TB:(i + 1) * TB]
        dl = jax.lax.iota(jnp.int32, TB).astype(jnp.float32)
        gt = dg[None, :] > st[:, None]
        ge = (dg[None, :] >= st[:, None]) & (dl[None, :] < dl[:, None])
        cnt = cnt + jnp.sum(jnp.where(gt | ge, 1.0, 0.0), axis=1)
        parts.append(cnt)
    rank = jnp.concatenate(parts)

    kidx = jax.lax.iota(jnp.int32, K).astype(jnp.float32)
    P = jnp.where(rank[None, :] == kidx[:, None], 1.0, 0.0)
    centers = jax.lax.dot_general(P, feat, (((1,), (0,)), ((), ())),
                                  preferred_element_type=jnp.float32)

    sims = jax.lax.dot_general(feat, centers, (((1,), (1,)), ((), ())),
                               preferred_element_type=jnp.float32)
    mx = jnp.max(sims, axis=1)
    am = jnp.min(jnp.where(sims == mx[:, None], kidx[None, :], float(K)),
                 axis=1)
    assign = jnp.where(rank < float(K), rank, am)

    A = jnp.where(assign[None, :] == kidx[:, None], 1.0, 0.0)
    sizes = jnp.sum(A, axis=1)
    inv_ref[0, 0] = 1.0 / jnp.clip(sizes, 1.0, None)
    assign_ref[0, 0] = assign.astype(jnp.int32)


def _tc_assign(x, base, nrm):
    B, T, C = x.shape
    K = _K
    return pl.pallas_call(
        _assign_body,
        grid=(B,),
        in_specs=[pl.BlockSpec((1, T, C), lambda b: (b, 0, 0)),
                  pl.BlockSpec((1, 1, T), lambda b: (b, 0, 0)),
                  pl.BlockSpec((1, 1, T), lambda b: (b, 0, 0))],
        out_specs=[pl.BlockSpec((1, 1, T), lambda b: (b, 0, 0)),
                   pl.BlockSpec((1, 1, K), lambda b: (b, 0, 0))],
        out_shape=[jax.ShapeDtypeStruct((B, 1, T), jnp.int32),
                   jax.ShapeDtypeStruct((B, 1, K), jnp.float32)],
    )(x, base.reshape(B, 1, T), nrm.reshape(B, 1, T))


_CSL = 128
_TCH = 256


def _merge_body(x_ref, aidx_ref, inv_ref, out_ref,
                rows_a, rows_b, idx_a, idx_b, acc_v, inv_v, sem_a, sem_b):
    C, K, T = _C, _K, _T
    c = lax.axis_index("c")
    s = lax.axis_index("s")
    w = c * 16 + s
    b = w // 6
    c0 = (w % 6) * _CSL

    @pl.when(w < 24)
    def _active():
        zv = jnp.zeros((16,), jnp.float32)

        def _zero_row(r, _):
            for v in range(_CSL // 16):
                acc_v[r, pl.ds(v * 16, 16)] = zv
            return 0

        lax.fori_loop(0, K, _zero_row, 0)

        def _start(i, rows, idx, sem):
            off = b * T + i * _TCH
            pltpu.async_copy(
                x_ref.at[pl.ds(off, _TCH), pl.ds(c0, _CSL)], rows, sem)
            pltpu.async_copy(aidx_ref.at[pl.ds(off, _TCH)], idx, sem)

        def _drain(i, rows, idx, sem):
            off = b * T + i * _TCH
            pltpu.make_async_copy(
                x_ref.at[pl.ds(off, _TCH), pl.ds(c0, _CSL)], rows, sem).wait()
            pltpu.make_async_copy(aidx_ref.at[pl.ds(off, _TCH)], idx, sem).wait()

        def _consume(rows, idx):
            def _tok16(g, _2):
                idxvec = idx[pl.ds(g * 16, 16)]
                ks = [idxvec[j] for j in range(16)]
                for j in range(16):
                    for v in range(_CSL // 16):
                        sl = pl.ds(v * 16, 16)
                        plsc.addupdate(acc_v.at[ks[j], sl],
                                       rows[g * 16 + j, sl])
                return 0

            lax.fori_loop(0, _TCH // 16, _tok16, 0)

        _start(0, rows_a, idx_a, sem_a)

        def _pair(p, _):
            i = 2 * p
            _start(i + 1, rows_b, idx_b, sem_b)
            _drain(i, rows_a, idx_a, sem_a)
            _consume(rows_a, idx_a)

            @pl.when(i + 2 < T // _TCH)
            def _nxt():
                _start(i + 2, rows_a, idx_a, sem_a)

            _drain(i + 1, rows_b, idx_b, sem_b)
            _consume(rows_b, idx_b)
            return 0

        lax.fori_loop(0, T // _TCH // 2, _pair, 0)

        pltpu.sync_copy(inv_ref.at[pl.ds(b * K, K)], inv_v)

        def _scale16(g, _):
            iv16 = inv_v[pl.ds(g * 16, 16)]
            for j in range(16):
                iv = jnp.full((16,), iv16[j], jnp.float32)
                r = g * 16 + j
                for v in range(_CSL // 16):
                    sl = pl.ds(v * 16, 16)
                    acc_v[r, sl] = acc_v[r, sl] * iv
            return 0

        lax.fori_loop(0, K // 16, _scale16, 0)
        pltpu.sync_copy(acc_v, out_ref.at[pl.ds(b * K, K), pl.ds(c0, _CSL)])


def _sc_merge(x_flat, aidx_flat, inv_flat):
    BT, C = x_flat.shape
    BK = inv_flat.shape[0]
    mesh = plsc.VectorSubcoreMesh(core_axis_name="c", subcore_axis_name="s")
    f = pl.kernel(
        _merge_body,
        out_type=jax.ShapeDtypeStruct((BK, C), jnp.float32),
        mesh=mesh,
        scratch_types=[
            pltpu.VMEM((_TCH, _CSL), jnp.float32),
            pltpu.VMEM((_TCH, _CSL), jnp.float32),
            pltpu.VMEM((_TCH,), jnp.int32),
            pltpu.VMEM((_TCH,), jnp.int32),
            pltpu.VMEM((256, _CSL), jnp.float32),
            pltpu.VMEM((256,), jnp.float32),
            pltpu.SemaphoreType.DMA,
            pltpu.SemaphoreType.DMA,
        ],
    )
    return f(x_flat, aidx_flat, inv_flat)


def kernel(x, layer_idx, requested_r):
    B, T, C = x.shape
    K = max(1, T - _REQ_R)
    sumsq = jnp.sum(x * x, axis=-1)
    base = jnp.sqrt(sumsq + 1e-6)
    nrm = jnp.clip(jnp.linalg.norm(x, axis=-1), 1e-12, None)
    assign3, inv3 = _tc_assign(x, base, nrm)
    out_flat = _sc_merge(x.reshape(B * T, C),
                         assign3.reshape(B * T),
                         inv3.reshape(B * K))
    return out_flat.reshape(B, K, C)
